# baseline (speedup 1.0000x reference)
import jax
import jax.numpy as jnp
from jax import lax
from jax.experimental import pallas as pl
from jax.experimental.pallas import tpu as pltpu
from jax.experimental.pallas import tpu_sc as plsc

L = 10
KNB = 4
F = 16
C = 256
G = 128
NC, NS = 2, 16
NW = NC * NS


def _splits(c_pts):
    return [(0, c_pts)]


def _make_sc_call(n_points):
    assert n_points % F == 0 and n_points >= C
    n_full = n_points // C
    tail = n_points % C
    n_chunks = n_full + (1 if tail else 0)
    iters = -(-n_chunks // NW)

    mesh = plsc.VectorSubcoreMesh(core_axis_name="c", subcore_axis_name="s")

    def body(idx_hbm, w_hbm, table_hbm, out_hbm,
             idx_v, w_v, rows_v, acc_v, sem_in, sem_g):
        wid = lax.axis_index("s") * NC + lax.axis_index("c")
        lanes = lax.iota(jnp.int32, F) * L

        def chunk_work(base, c_pts):
            cp_i = pltpu.async_copy(
                idx_hbm.at[:, :, pl.ds(base, c_pts)],
                idx_v.at[:, :, pl.ds(0, c_pts)], sem_in)
            cp_w = pltpu.async_copy(
                w_hbm.at[:, :, pl.ds(base, c_pts)],
                w_v.at[:, :, pl.ds(0, c_pts)], sem_in)
            cp_i.wait()
            cp_w.wait()

            def fire(l):
                buf = (l % 2) * KNB * C
                for j in range(KNB):
                    for off, sz in _splits(c_pts):
                        pltpu.async_copy(
                            table_hbm.at[idx_v.at[l, j, pl.ds(off, sz)]],
                            rows_v.at[pl.ds(buf + j * C + off, sz)], sem_g)

            def level_body(l, carry2):
                @pl.when(l < L)
                def _fire():
                    fire(l)

                @pl.when(l > 0)
                def _compute():
                    lp = l - 1
                    buf = (lp % 2) * KNB * C
                    pltpu.make_async_copy(
                        table_hbm.at[pl.ds(0, KNB * c_pts)],
                        rows_v.at[pl.ds(buf, KNB * c_pts)], sem_g).wait()
                    lane_l = lanes + lp

                    def grp_body(g, carry3):
                        n0 = g * F
                        wv = [w_v[lp, j, pl.ds(n0, F)] for j in range(KNB)]
                        for i in range(F):
                            nn = n0 + i
                            acc = rows_v[buf + 0 * C + nn] * wv[0][i]
                            acc = acc + rows_v[buf + 1 * C + nn] * wv[1][i]
                            acc = acc + rows_v[buf + 2 * C + nn] * wv[2][i]
                            acc = acc + rows_v[buf + 3 * C + nn] * wv[3][i]
                            plsc.store_scatter(
                                acc_v,
                                [jnp.full((F,), nn, jnp.int32), lane_l], acc)
                        return carry3

                    lax.fori_loop(0, c_pts // F, grp_body, 0)
                return carry2

            lax.fori_loop(0, L + 1, level_body, 0)
            pltpu.sync_copy(acc_v.at[pl.ds(0, c_pts)],
                            out_hbm.at[pl.ds(base, c_pts)])

        def chunk_body(t, carry):
            k = t * NW + wid

            @pl.when(k < n_full)
            def _full():
                chunk_work(k * C, C)

            if tail:
                @pl.when(k == n_full)
                def _tail():
                    chunk_work(n_full * C, tail)
            return carry

        lax.fori_loop(0, iters, chunk_body, 0)

    return pl.kernel(
        body,
        out_type=jax.ShapeDtypeStruct((n_points, F * L), jnp.float32),
        mesh=mesh,
        compiler_params=pltpu.CompilerParams(
            needs_layout_passes=False, use_tc_tiling_on_sc=False),
        scratch_types=[
            pltpu.VMEM((L, KNB, C), jnp.int32),
            pltpu.VMEM((L, KNB, C), jnp.float32),
            pltpu.VMEM((2 * KNB * C, F), jnp.float32),
            pltpu.VMEM((C, F * L), jnp.float32),
            pltpu.SemaphoreType.DMA,
            pltpu.SemaphoreType.DMA,
        ],
    )


def kernel(x, params, neigh_pix, neigh_weight):
    n = x.shape[0]
    run = _make_sc_call(n)
    return run(neigh_pix, neigh_weight, params)


# speedup vs baseline: 1.2558x; 1.2558x over previous
import jax
import jax.numpy as jnp
from jax import lax
from jax.experimental import pallas as pl
from jax.experimental.pallas import tpu as pltpu
from jax.experimental.pallas import tpu_sc as plsc

L = 10
LL = 5
SL = L - LL
KNB = 4
F = 16
C = 128
TAB = 4096
NC, NS = 2, 16
NW = NC * NS


def _make_sc_call(n_points):
    assert n_points % F == 0 and n_points >= C
    n_full = n_points // C
    tail = n_points % C
    n_chunks = n_full + (1 if tail else 0)
    iters = -(-n_chunks // NW)

    mesh = plsc.VectorSubcoreMesh(core_axis_name="c", subcore_axis_name="s")

    def body(idx_hbm, w_hbm, table_hbm, out_hbm,
             idx_v, w_v, tab_v, rows_v, acc_v, sem_in, sem_g):
        wid = lax.axis_index("s") * NC + lax.axis_index("c")
        lanes = lax.iota(jnp.int32, F) * L
        iota = lax.iota(jnp.int32, F)

        pltpu.sync_copy(table_hbm.at[pl.ds(0, TAB)], tab_v)

        def chunk_work(base, c_pts):
            cp_i = pltpu.async_copy(
                idx_hbm.at[:, :, pl.ds(base, c_pts)],
                idx_v.at[:, :, pl.ds(0, c_pts)], sem_in)
            cp_w = pltpu.async_copy(
                w_hbm.at[:, :, pl.ds(base, c_pts)],
                w_v.at[:, :, pl.ds(0, c_pts)], sem_in)
            cp_i.wait()
            cp_w.wait()

            def fire(l):
                buf = (l % 2) * KNB * C
                for j in range(KNB):
                    pltpu.async_copy(
                        table_hbm.at[idx_v.at[l, j, pl.ds(0, c_pts)]],
                        rows_v.at[pl.ds(buf + j * C, c_pts)], sem_g)

            def level_body(ls, carry2):
                @pl.when(ls < SL)
                def _fire():
                    fire(LL + ls)

                @pl.when(ls < LL)
                def _local():
                    lane_l = lanes + ls

                    def lgrp_body(g, carry3):
                        n0 = g * F
                        rowsel = n0 + iota
                        rv = [idx_v[ls, j, pl.ds(n0, F)] for j in range(KNB)]
                        wv = [w_v[ls, j, pl.ds(n0, F)] for j in range(KNB)]
                        for f in range(F):
                            colf = jnp.full((F,), f, jnp.int32)
                            gj = [plsc.load_gather(tab_v, [rv[j], colf])
                                  for j in range(KNB)]
                            accv = gj[0] * wv[0]
                            accv = accv + gj[1] * wv[1]
                            accv = accv + gj[2] * wv[2]
                            accv = accv + gj[3] * wv[3]
                            plsc.store_scatter(
                                acc_v, [rowsel, jnp.full((F,), f * L, jnp.int32) + ls],
                                accv)
                        return carry3

                    lax.fori_loop(0, c_pts // F, lgrp_body, 0)

                @pl.when(ls > 0)
                def _stream():
                    lp = LL + ls - 1
                    buf = (lp % 2) * KNB * C
                    pltpu.make_async_copy(
                        table_hbm.at[pl.ds(0, KNB * c_pts)],
                        rows_v.at[pl.ds(buf, KNB * c_pts)], sem_g).wait()
                    lane_l = lanes + lp

                    def grp_body(g, carry3):
                        n0 = g * F
                        wv = [w_v[lp, j, pl.ds(n0, F)] for j in range(KNB)]
                        for i in range(F):
                            nn = n0 + i
                            acc = rows_v[buf + 0 * C + nn] * wv[0][i]
                            acc = acc + rows_v[buf + 1 * C + nn] * wv[1][i]
                            acc = acc + rows_v[buf + 2 * C + nn] * wv[2][i]
                            acc = acc + rows_v[buf + 3 * C + nn] * wv[3][i]
                            plsc.store_scatter(
                                acc_v,
                                [jnp.full((F,), nn, jnp.int32), lane_l], acc)
                        return carry3

                    lax.fori_loop(0, c_pts // F, grp_body, 0)
                return carry2

            lax.fori_loop(0, SL + 1, level_body, 0)
            pltpu.sync_copy(acc_v.at[pl.ds(0, c_pts)],
                            out_hbm.at[pl.ds(base, c_pts)])

        def chunk_body(t, carry):
            k = t * NW + wid

            @pl.when(k < n_full)
            def _full():
                chunk_work(k * C, C)

            if tail:
                @pl.when(k == n_full)
                def _tail():
                    chunk_work(n_full * C, tail)
            return carry

        lax.fori_loop(0, iters, chunk_body, 0)

    return pl.kernel(
        body,
        out_type=jax.ShapeDtypeStruct((n_points, F * L), jnp.float32),
        mesh=mesh,
        compiler_params=pltpu.CompilerParams(
            needs_layout_passes=False, use_tc_tiling_on_sc=False),
        scratch_types=[
            pltpu.VMEM((L, KNB, C), jnp.int32),
            pltpu.VMEM((L, KNB, C), jnp.float32),
            pltpu.VMEM((TAB, F), jnp.float32),
            pltpu.VMEM((2 * KNB * C, F), jnp.float32),
            pltpu.VMEM((C, F * L), jnp.float32),
            pltpu.SemaphoreType.DMA,
            pltpu.SemaphoreType.DMA,
        ],
    )


def kernel(x, params, neigh_pix, neigh_weight):
    n = x.shape[0]
    idx2 = jnp.bitwise_and(neigh_pix, jnp.int32(0x7FFFFFFF))
    w2 = jnp.abs(neigh_weight)
    run = _make_sc_call(n)
    return run(idx2, w2, params)


# speedup vs baseline: 1.2645x; 1.0069x over previous
import jax
import jax.numpy as jnp
from jax import lax
from jax.experimental import pallas as pl
from jax.experimental.pallas import tpu as pltpu
from jax.experimental.pallas import tpu_sc as plsc

L = 10
LL = 5
SL = L - LL
KNB = 4
F = 16
C = 128
TAB = 4096
NC, NS = 2, 16
NW = NC * NS


def _make_sc_call(n_points):
    assert n_points % F == 0 and n_points >= C
    n_full = n_points // C
    tail = n_points % C
    n_chunks = n_full + (1 if tail else 0)
    iters = -(-n_chunks // NW)

    mesh = plsc.VectorSubcoreMesh(core_axis_name="c", subcore_axis_name="s")

    def body(idx_hbm, w_hbm, table_hbm, out_hbm,
             idx_v, w_v, tab_v, rows_v, acc_v, sem_in, sem_g):
        wid = lax.axis_index("s") * NC + lax.axis_index("c")
        lanes = lax.iota(jnp.int32, F) * L
        iota = lax.iota(jnp.int32, F)

        pltpu.sync_copy(table_hbm.at[pl.ds(0, TAB)], tab_v)

        def chunk_work(base, c_pts):
            cp_i = pltpu.async_copy(
                idx_hbm.at[:, :, pl.ds(base, c_pts)],
                idx_v.at[:, :, pl.ds(0, c_pts)], sem_in)
            cp_w = pltpu.async_copy(
                w_hbm.at[:, :, pl.ds(base, c_pts)],
                w_v.at[:, :, pl.ds(0, c_pts)], sem_in)
            cp_i.wait()
            cp_w.wait()

            def fire(l):
                buf = (l % 2) * KNB * C
                for j in range(KNB):
                    pltpu.async_copy(
                        table_hbm.at[idx_v.at[l, j, pl.ds(0, c_pts)]],
                        rows_v.at[pl.ds(buf + j * C, c_pts)], sem_g)

            def combine_streamed(lp, buf):
                lane_l = lanes + lp

                def grp_body(g, carry3):
                    n0 = g * F
                    wv = [w_v[lp, j, pl.ds(n0, F)] for j in range(KNB)]
                    for i in range(F):
                        nn = n0 + i
                        acc = rows_v[buf + 0 * C + nn] * wv[0][i]
                        acc = acc + rows_v[buf + 1 * C + nn] * wv[1][i]
                        acc = acc + rows_v[buf + 2 * C + nn] * wv[2][i]
                        acc = acc + rows_v[buf + 3 * C + nn] * wv[3][i]
                        plsc.store_scatter(
                            acc_v,
                            [jnp.full((F,), nn, jnp.int32), lane_l], acc)
                    return carry3

                lax.fori_loop(0, c_pts // F, grp_body, 0)

            def level_body(ls, carry2):
                @pl.when(ls < SL)
                def _fire():
                    fire(LL + ls)

                @pl.when(ls < LL)
                def _local():
                    def lgrp_body(g, carry3):
                        n0 = g * F
                        rowsel = n0 + iota
                        rv = [idx_v[ls, j, pl.ds(n0, F)] for j in range(KNB)]
                        wv = [w_v[ls, j, pl.ds(n0, F)] for j in range(KNB)]
                        for f in range(F):
                            colf = jnp.full((F,), f, jnp.int32)
                            gj = [plsc.load_gather(tab_v, [rv[j], colf])
                                  for j in range(KNB)]
                            accv = gj[0] * wv[0]
                            accv = accv + gj[1] * wv[1]
                            accv = accv + gj[2] * wv[2]
                            accv = accv + gj[3] * wv[3]
                            plsc.store_scatter(
                                acc_v,
                                [rowsel, jnp.full((F,), f * L, jnp.int32) + ls],
                                accv)
                        return carry3

                    lax.fori_loop(0, c_pts // F, lgrp_body, 0)

                @pl.when(ls > 0)
                def _stream():
                    lp = LL + ls - 1
                    buf = (lp % 2) * KNB * C
                    pltpu.make_async_copy(
                        table_hbm.at[pl.ds(0, KNB * c_pts)],
                        rows_v.at[pl.ds(buf, KNB * c_pts)], sem_g).wait()
                    combine_streamed(lp, buf)
                return carry2

            lax.fori_loop(0, SL + 1, level_body, 0)
            pltpu.sync_copy(acc_v.at[pl.ds(0, c_pts)],
                            out_hbm.at[pl.ds(base, c_pts)])

        def chunk_body(t, carry):
            k = t * NW + wid

            @pl.when(k < n_full)
            def _full():
                chunk_work(k * C, C)

            if tail:
                @pl.when(k == n_full)
                def _tail():
                    chunk_work(n_full * C, tail)
            return carry

        lax.fori_loop(0, iters, chunk_body, 0)

    return pl.kernel(
        body,
        out_type=jax.ShapeDtypeStruct((n_points, F * L), jnp.float32),
        mesh=mesh,
        compiler_params=pltpu.CompilerParams(
            needs_layout_passes=False, use_tc_tiling_on_sc=False),
        scratch_types=[
            pltpu.VMEM((L, KNB, C), jnp.int32),
            pltpu.VMEM((L, KNB, C), jnp.float32),
            pltpu.VMEM((TAB, F), jnp.float32),
            pltpu.VMEM((2 * KNB * C, F), jnp.float32),
            pltpu.VMEM((C, F * L), jnp.float32),
            pltpu.SemaphoreType.DMA,
            pltpu.SemaphoreType.DMA,
        ],
    )


def kernel(x, params, neigh_pix, neigh_weight):
    n = x.shape[0]
    run = _make_sc_call(n)
    return run(neigh_pix, neigh_weight, params)


# speedup vs baseline: 1.2768x; 1.0098x over previous
import jax
import jax.numpy as jnp
from jax import lax
from jax.experimental import pallas as pl
from jax.experimental.pallas import tpu as pltpu
from jax.experimental.pallas import tpu_sc as plsc

L = 10
LL = 5
SL = L - LL
KNB = 4
F = 16
C = 128
TAB = 4096
NC, NS = 2, 16
NW = NC * NS


def _make_sc_call(n_points):
    assert n_points % 8 == 0 and n_points % F == 0 and n_points >= C
    n_chunks = -(-n_points // C)
    iters = -(-n_chunks // NW)
    last_base = n_points - C

    mesh = plsc.VectorSubcoreMesh(core_axis_name="c", subcore_axis_name="s")

    def body(idx_hbm, w_hbm, table_hbm, out_hbm,
             idx_v, w_v, tab_v, rows_v, acc_v, sem_in, sem_g):
        wid = lax.axis_index("s") * NC + lax.axis_index("c")
        lanes = lax.iota(jnp.int32, F) * L
        iota = lax.iota(jnp.int32, F)

        pltpu.sync_copy(table_hbm.at[pl.ds(0, TAB)], tab_v)

        def stage(k, par):
            base = jnp.minimum(k * C, last_base)
            pltpu.async_copy(idx_hbm.at[:, :, pl.ds(base, C)],
                             idx_v.at[par], sem_in)
            pltpu.async_copy(w_hbm.at[:, :, pl.ds(base, C)],
                             w_v.at[par], sem_in)

        stage(wid, 0)

        def chunk_work(t, k, par):
            base = jnp.minimum(k * C, last_base)
            pltpu.make_async_copy(
                idx_hbm.at[:, :, pl.ds(0, C)], idx_v.at[par], sem_in).wait()
            pltpu.make_async_copy(
                w_hbm.at[:, :, pl.ds(0, C)], w_v.at[par], sem_in).wait()

            nk = k + NW
            @pl.when(nk < n_chunks)
            def _prefetch():
                stage(nk, 1 - par)

            def fire(l):
                buf = (l % 2) * KNB * C
                for j in range(KNB):
                    pltpu.async_copy(
                        table_hbm.at[idx_v.at[par, l, j, pl.ds(0, C)]],
                        rows_v.at[pl.ds(buf + j * C, C)], sem_g)

            def combine_streamed(lp, buf):
                lane_l = lanes + lp

                def grp_body(g, carry3):
                    n0 = g * F
                    wv = [w_v[par, lp, j, pl.ds(n0, F)] for j in range(KNB)]
                    for i in range(F):
                        nn = n0 + i
                        acc = rows_v[buf + 0 * C + nn] * wv[0][i]
                        acc = acc + rows_v[buf + 1 * C + nn] * wv[1][i]
                        acc = acc + rows_v[buf + 2 * C + nn] * wv[2][i]
                        acc = acc + rows_v[buf + 3 * C + nn] * wv[3][i]
                        plsc.store_scatter(
                            acc_v,
                            [jnp.full((F,), nn, jnp.int32), lane_l], acc)
                    return carry3

                lax.fori_loop(0, C // F, grp_body, 0)

            def level_body(ls, carry2):
                @pl.when(ls < SL)
                def _fire():
                    fire(LL + ls)

                @pl.when(ls < LL)
                def _local():
                    def lgrp_body(g, carry3):
                        n0 = g * F
                        rowsel = n0 + iota
                        rv = [idx_v[par, ls, j, pl.ds(n0, F)]
                              for j in range(KNB)]
                        wv = [w_v[par, ls, j, pl.ds(n0, F)]
                              for j in range(KNB)]
                        for f in range(F):
                            colf = jnp.full((F,), f, jnp.int32)
                            gj = [plsc.load_gather(tab_v, [rv[j], colf])
                                  for j in range(KNB)]
                            accv = gj[0] * wv[0]
                            accv = accv + gj[1] * wv[1]
                            accv = accv + gj[2] * wv[2]
                            accv = accv + gj[3] * wv[3]
                            plsc.store_scatter(
                                acc_v,
                                [rowsel,
                                 jnp.full((F,), f * L, jnp.int32) + ls],
                                accv)
                        return carry3

                    lax.fori_loop(0, C // F, lgrp_body, 0)

                @pl.when(ls > 0)
                def _stream():
                    lp = LL + ls - 1
                    buf = (lp % 2) * KNB * C
                    pltpu.make_async_copy(
                        table_hbm.at[pl.ds(0, KNB * C)],
                        rows_v.at[pl.ds(buf, KNB * C)], sem_g).wait()
                    combine_streamed(lp, buf)
                return carry2

            lax.fori_loop(0, SL + 1, level_body, 0)
            pltpu.sync_copy(acc_v, out_hbm.at[pl.ds(base, C)])

        def chunk_body(t, carry):
            k = t * NW + wid

            @pl.when(k < n_chunks)
            def _go():
                chunk_work(t, k, t % 2)
            return carry

        lax.fori_loop(0, iters, chunk_body, 0)

    return pl.kernel(
        body,
        out_type=jax.ShapeDtypeStruct((n_points, F * L), jnp.float32),
        mesh=mesh,
        compiler_params=pltpu.CompilerParams(
            needs_layout_passes=False, use_tc_tiling_on_sc=False),
        scratch_types=[
            pltpu.VMEM((2, L, KNB, C), jnp.int32),
            pltpu.VMEM((2, L, KNB, C), jnp.float32),
            pltpu.VMEM((TAB, F), jnp.float32),
            pltpu.VMEM((2 * KNB * C, F), jnp.float32),
            pltpu.VMEM((C, F * L), jnp.float32),
            pltpu.SemaphoreType.DMA,
            pltpu.SemaphoreType.DMA,
        ],
    )


def kernel(x, params, neigh_pix, neigh_weight):
    n = x.shape[0]
    run = _make_sc_call(n)
    return run(neigh_pix, neigh_weight, params)
